# Initial kernel scaffold; baseline (speedup 1.0000x reference)
import jax
import jax.numpy as jnp
from jax.experimental import pallas as pl


def kernel(indices, table):
    raise NotImplementedError("write your pallas kernel here")


# speedup vs baseline: 2.4443x; 2.4443x over previous
import functools

import jax
import jax.numpy as jnp
from jax import lax
from jax.experimental import pallas as pl
from jax.experimental.pallas import tpu as pltpu
from jax.experimental.pallas import tpu_sc as plsc

VOCAB = 1000000
D = 64
B = 16384
H = 50

NC = 2
NS = 16
NW = NC * NS
BPW = B // NW

BAGS_PER_GATHER = 2
IDX_PER_GATHER = BAGS_PER_GATHER * H
GATHERS = 8
CB = BAGS_PER_GATHER * GATHERS
IDX_PER_CHUNK = CB * H
NCHUNK = BPW // CB
NLG = D // 16


def _embed_bag_body(idx_hbm, table_hbm, out_hbm, idx_v, rows_v, out_v, sem):
    wid = lax.axis_index("s") * NC + lax.axis_index("c")
    grow_base = wid * (BPW * H // IDX_PER_GATHER)
    row_base = wid * BPW

    def chunk_body(c, _):
        grow0 = grow_base + c * GATHERS
        row0 = row_base + c * CB
        pltpu.sync_copy(idx_hbm.at[pl.ds(grow0, GATHERS)], idx_v)
        cps = [
            pltpu.async_copy(table_hbm.at[idx_v.at[g]], rows_v.at[g], sem)
            for g in range(GATHERS)
        ]
        for cp in cps:
            cp.wait()

        for g in range(GATHERS):
            for j in range(BAGS_PER_GATHER):
                def row_body(r, accs, g=g, j=j):
                    return tuple(
                        accs[lg] + rows_v[g, j * H + r, pl.ds(lg * 16, 16)]
                        for lg in range(NLG)
                    )

                accs = lax.fori_loop(
                    0, H, row_body,
                    tuple(jnp.zeros((16,), jnp.float32) for _ in range(NLG)),
                )
                for lg in range(NLG):
                    out_v[g * BAGS_PER_GATHER + j, pl.ds(lg * 16, 16)] = (
                        accs[lg] * (1.0 / H)
                    )

        pltpu.sync_copy(out_v, out_hbm.at[pl.ds(row0, CB)])
        return 0

    lax.fori_loop(0, NCHUNK, chunk_body, 0)


@jax.jit
def kernel(indices, table):
    idx_rows = indices.reshape(-1, IDX_PER_GATHER)
    mesh = plsc.VectorSubcoreMesh(
        core_axis_name="c", subcore_axis_name="s", num_cores=NC, num_subcores=NS
    )
    k = functools.partial(
        pl.kernel,
        out_type=jax.ShapeDtypeStruct((B, D), jnp.float32),
        mesh=mesh,
        compiler_params=pltpu.CompilerParams(use_tc_tiling_on_sc=False),
        scratch_types=[
            pltpu.VMEM((GATHERS, IDX_PER_GATHER), jnp.int32),
            pltpu.VMEM((GATHERS, IDX_PER_GATHER, D), jnp.float32),
            pltpu.VMEM((CB, D), jnp.float32),
            pltpu.SemaphoreType.DMA,
        ],
    )(_embed_bag_body)
    return k(idx_rows, table)
